# baseline (speedup 1.0000x reference)
import jax
import jax.numpy as jnp
from jax.experimental import pallas as pl

_B = 16384
_TD = 32
_H = 768
_BLK = 2048


def _encoder_kernel(t_ref, c_ref, w_ref, b_ref, o_ref):
    combined = jnp.concatenate([t_ref[:], c_ref[:]], axis=-1)
    small = (
        jax.lax.dot_general(
            combined,
            w_ref[:],
            dimension_numbers=(((1,), (1,)), ((), ())),
            preferred_element_type=jnp.float32,
        )
        + b_ref[:]
    )
    o_ref[:] = jnp.broadcast_to(small[None], (_BLK, 4, _H))


def kernel(batch_size, temporal_embed, causal_embed, W, b):
    return pl.pallas_call(
        _encoder_kernel,
        grid=(_B // _BLK,),
        in_specs=[
            pl.BlockSpec((4, _TD), lambda i: (0, 0)),
            pl.BlockSpec((4, _TD), lambda i: (0, 0)),
            pl.BlockSpec((_H, 2 * _TD), lambda i: (0, 0)),
            pl.BlockSpec((1, _H), lambda i: (0, 0)),
        ],
        out_specs=pl.BlockSpec((_BLK, 4, _H), lambda i: (i, 0, 0)),
        out_shape=jax.ShapeDtypeStruct((_B, 4, _H), jnp.float32),
    )(temporal_embed, causal_embed, W, b.reshape(1, _H))


# speedup vs baseline: 1.0295x; 1.0295x over previous
import jax
import jax.numpy as jnp
from jax.experimental import pallas as pl

_B = 16384
_TD = 32
_H = 768
_BLK = 512


def _encoder_kernel(t_ref, c_ref, w_ref, b_ref, o_ref):
    combined = jnp.concatenate([t_ref[:], c_ref[:]], axis=-1)
    small = (
        jax.lax.dot_general(
            combined,
            w_ref[:],
            dimension_numbers=(((1,), (1,)), ((), ())),
            preferred_element_type=jnp.float32,
        )
        + b_ref[:]
    )
    o_ref[:] = jnp.broadcast_to(small[None], (_BLK, 4, _H))


def kernel(batch_size, temporal_embed, causal_embed, W, b):
    return pl.pallas_call(
        _encoder_kernel,
        grid=(_B // _BLK,),
        in_specs=[
            pl.BlockSpec((4, _TD), lambda i: (0, 0)),
            pl.BlockSpec((4, _TD), lambda i: (0, 0)),
            pl.BlockSpec((_H, 2 * _TD), lambda i: (0, 0)),
            pl.BlockSpec((1, _H), lambda i: (0, 0)),
        ],
        out_specs=pl.BlockSpec((_BLK, 4, _H), lambda i: (i, 0, 0)),
        out_shape=jax.ShapeDtypeStruct((_B, 4, _H), jnp.float32),
    )(temporal_embed, causal_embed, W, b.reshape(1, _H))
